# baseline (speedup 1.0000x reference)
import jax
import jax.numpy as jnp
from jax import lax
from jax.experimental import pallas as pl
from jax.experimental.pallas import tpu as pltpu
from jax.experimental.pallas import tpu_sc as plsc


def _sc_geometry():
    try:
        info = plsc.get_sparse_core_info()
        return info.num_cores, info.num_subcores, info.num_lanes
    except Exception:
        return 2, 16, 16


def _build_sc_call(V, H, N, VP):
    NC, NS, L = _sc_geometry()
    NW = NC * NS
    S = N + 1
    R = 8
    NGRP = N // R
    GRP_PER_W = NGRP // NW
    BLKS = N // L
    WP = (S // L + 1) * L

    mesh = plsc.VectorSubcoreMesh(core_axis_name="c", subcore_axis_name="s",
                                  num_cores=NC, num_subcores=NS)

    def body(tab_hbm, idxp_hbm, out_hbm, tab_v, idx_v, ob0, ob1, sem0, sem1):
        c = lax.axis_index("c")
        s = lax.axis_index("s")
        wid = s * NC + c

        pltpu.sync_copy(tab_hbm, tab_v)

        zf = jnp.zeros((L,), jnp.float32)
        lane = lax.iota(jnp.int32, L)
        m01 = jnp.where(lane == 0, 0.0, 1.0)
        lane0 = lane == 0
        colN = jnp.full((L,), N, jnp.int32)

        def fill_row(buf, r, hvec):
            v0 = plsc.load_gather(tab_v, [hvec, idx_v[r, pl.ds(0, L)]])
            buf[r, pl.ds(0, L)] = v0 * m01

            @plsc.parallel_loop(1, BLKS, unroll=8)
            def blk(j):
                vals = plsc.load_gather(tab_v,
                                        [hvec, idx_v[r, pl.ds(j * L, L)]])
                buf[r, pl.ds(j * L, L)] = vals
            vN = plsc.load_gather(tab_v, [hvec, idx_v[r, pl.ds(N, L)]])
            rvec = jnp.full((L,), 0, jnp.int32) + r
            plsc.store_scatter(buf, [rvec, colN], vN, mask=lane0)

        def zero_row0(buf):
            def zblk(i, carry):
                buf[0, pl.ds(i * L, L)] = zf
                return carry

            lax.fori_loop(0, BLKS, zblk, 0)
            plsc.store_scatter(buf, [jnp.full((L,), 0, jnp.int32), colN],
                               zf, mask=lane0)

        @pl.when(wid < H)
        def _tail():
            pltpu.sync_copy(idxp_hbm.at[pl.ds(N, 1)],
                            idx_v.at[pl.ds(0, 1)])
            fill_row(ob0, 0, jnp.full((L,), 0, jnp.int32) + wid)
            pltpu.sync_copy(ob0.at[pl.ds(0, 1)],
                            out_hbm.at[wid, pl.ds(N, 1)])

        obufs = (ob0, ob1)
        sems = (sem0, sem1)
        pending = [None, None]
        t = 0
        for kk in range(GRP_PER_W):
            k = wid * GRP_PER_W + kk
            ro = pl.multiple_of(k * R, R)
            pltpu.sync_copy(idxp_hbm.at[pl.ds(ro, R)], idx_v)
            for h in range(H):
                p = t % 2
                if pending[p] is not None:
                    pending[p].wait()
                buf = obufs[p]
                hvec = jnp.full((L,), h, jnp.int32)

                def rows(r, carry, buf=buf, hvec=hvec):
                    fill_row(buf, r, hvec)
                    return carry

                lax.fori_loop(0, R, rows, 0)
                if kk == 0:
                    @pl.when(wid == 0)
                    def _z(buf=buf):
                        zero_row0(buf)
                pending[p] = pltpu.async_copy(
                    buf, out_hbm.at[h, pl.ds(ro, R)], sems[p])
                t += 1
        for p in range(2):
            if pending[p] is not None:
                pending[p].wait()

    call = pl.kernel(
        body,
        out_type=jax.ShapeDtypeStruct((H, S, S), jnp.float32),
        mesh=mesh,
        compiler_params=pltpu.CompilerParams(needs_layout_passes=False),
        scratch_types=[
            pltpu.VMEM((H, VP), jnp.float32),
            pltpu.VMEM((R, WP), jnp.int32),
            pltpu.VMEM((R, S), jnp.float32),
            pltpu.VMEM((R, S), jnp.float32),
            pltpu.SemaphoreType.DMA,
            pltpu.SemaphoreType.DMA,
        ],
    )
    return call


def _pad_inputs_tc(table, idx, VP, WP):
    V, H = table.shape
    N = idx.shape[0]
    S = N + 1

    def body(tab_ref, idx_ref, tabo_ref, idxo_ref):
        tabo_ref[...] = jnp.pad(tab_ref[...].T, ((0, 0), (0, VP - V)))
        idxo_ref[...] = jnp.pad(idx_ref[...], ((1, 0), (1, WP - S)))

    return pl.pallas_call(
        body,
        out_shape=(
            jax.ShapeDtypeStruct((H, VP), jnp.float32),
            jax.ShapeDtypeStruct((S, WP), jnp.int32),
        ),
    )(table, idx)


def kernel(relative_position_bias_table, relative_position_index, seq_len):
    V, H = relative_position_bias_table.shape
    N = relative_position_index.shape[0]
    S = N + 1
    L = 16
    WP = (S // L + 1) * L
    VP = -(-V // 128) * 128
    idx = relative_position_index.astype(jnp.int32)
    tab_t, idx_pad = _pad_inputs_tc(relative_position_bias_table, idx,
                                    VP, WP)
    call = _build_sc_call(V, H, N, VP)
    out = call(tab_t, idx_pad)
    return out[None]


# speedup vs baseline: 1.7805x; 1.7805x over previous
import jax
import jax.numpy as jnp
from jax import lax
from jax.experimental import pallas as pl
from jax.experimental.pallas import tpu as pltpu
from jax.experimental.pallas import tpu_sc as plsc


def _sc_geometry():
    try:
        info = plsc.get_sparse_core_info()
        return info.num_cores, info.num_subcores, info.num_lanes
    except Exception:
        return 2, 16, 16


def _build_sc_call(V, H, N, VP):
    NC, NS, L = _sc_geometry()
    NW = NC * NS
    S = N + 1
    RPW = N // NW
    RCHUNK = 8
    NM = RPW // RCHUNK
    BLKS = N // L
    WP = (S // L + 1) * L

    mesh = plsc.VectorSubcoreMesh(core_axis_name="c", subcore_axis_name="s",
                                  num_cores=NC, num_subcores=NS)

    def body(tab_hbm, idxp_hbm, out_hbm, tab_v, idx_v, ob0, ob1, sem0, sem1):
        c = lax.axis_index("c")
        s = lax.axis_index("s")
        wid = s * NC + c

        pltpu.sync_copy(tab_hbm, tab_v)

        zf = jnp.zeros((L,), jnp.float32)
        z16 = jnp.zeros((L,), jnp.int32)
        lane = lax.iota(jnp.int32, L)
        m01 = jnp.where(lane == 0, 0.0, 1.0)
        lane0 = lane == 0
        colN = jnp.full((L,), N, jnp.int32)

        def fill_buf(buf, rr):
            def hloop(h, carry):
                tabh = tab_v.at[pl.ds(pl.multiple_of(h * VP, 128), VP)]
                hv = z16 + h
                v0 = plsc.load_gather(tabh, [idx_v[rr, pl.ds(0, L)]])
                buf[0, h, pl.ds(0, L)] = v0 * m01

                @plsc.parallel_loop(1, BLKS, unroll=8)
                def blk(j):
                    vals = plsc.load_gather(tabh,
                                            [idx_v[rr, pl.ds(j * L, L)]])
                    buf[0, h, pl.ds(j * L, L)] = vals

                vN = plsc.load_gather(tabh, [idx_v[rr, pl.ds(N, L)]])
                plsc.store_scatter(buf, [z16, hv, colN], vN, mask=lane0)
                return carry

            lax.fori_loop(0, H, hloop, 0)

        def zero_buf(buf):
            def hloop(h, carry):
                def zblk(j, inner):
                    buf[0, h, pl.ds(j * L, L)] = zf
                    return inner

                lax.fori_loop(0, BLKS, zblk, 0)
                plsc.store_scatter(buf, [z16, z16 + h, colN], zf,
                                   mask=lane0)
                return carry

            lax.fori_loop(0, H, hloop, 0)

        obufs = (ob0, ob1)
        sems = (sem0, sem1)
        pending = [None, None]
        t = 0
        base = wid * RPW
        for m in range(NM):
            i0 = pl.multiple_of(base + m * RCHUNK, RCHUNK)
            pltpu.sync_copy(idxp_hbm.at[pl.ds(i0, RCHUNK)], idx_v)
            for rb in range(RCHUNK):
                p = t % 2
                if pending[p] is not None:
                    pending[p].wait()
                buf = obufs[p]
                fill_buf(buf, rb)
                if m == 0 and rb == 0:
                    @pl.when(wid == 0)
                    def _z(buf=buf):
                        zero_buf(buf)
                pending[p] = pltpu.async_copy(
                    buf, out_hbm.at[pl.ds(i0 + rb, 1)], sems[p])
                t += 1
        for p in range(2):
            if pending[p] is not None:
                pending[p].wait()

        @pl.when(wid == NW - 1)
        def _tail():
            pltpu.sync_copy(idxp_hbm.at[pl.ds(N, 1)],
                            idx_v.at[pl.ds(0, 1)])
            fill_buf(ob0, 0)
            pltpu.sync_copy(ob0, out_hbm.at[pl.ds(N, 1)])

    call = pl.kernel(
        body,
        out_type=jax.ShapeDtypeStruct((S, H, S), jnp.float32),
        mesh=mesh,
        compiler_params=pltpu.CompilerParams(needs_layout_passes=False),
        scratch_types=[
            pltpu.VMEM((H * VP,), jnp.float32),
            pltpu.VMEM((RCHUNK, WP), jnp.int32),
            pltpu.VMEM((1, H, S), jnp.float32),
            pltpu.VMEM((1, H, S), jnp.float32),
            pltpu.SemaphoreType.DMA,
            pltpu.SemaphoreType.DMA,
        ],
    )
    return call


def kernel(relative_position_bias_table, relative_position_index, seq_len):
    V, H = relative_position_bias_table.shape
    N = relative_position_index.shape[0]
    S = N + 1
    L = 16
    WP = (S // L + 1) * L
    VP = -(-V // 128) * 128
    tab_flat = jnp.pad(relative_position_bias_table.T,
                       ((0, 0), (0, VP - V))).reshape(-1)
    idx = relative_position_index.astype(jnp.int32)
    idx_pad = jnp.pad(idx, ((1, 0), (1, WP - S)))
    call = _build_sc_call(V, H, N, VP)
    out = call(tab_flat, idx_pad)
    return jnp.transpose(out, (1, 0, 2))[None]


# speedup vs baseline: 2.9993x; 1.6846x over previous
import jax
import jax.numpy as jnp
from jax import lax
from jax.experimental import pallas as pl
from jax.experimental.pallas import tpu as pltpu
from jax.experimental.pallas import tpu_sc as plsc


def _sc_geometry():
    try:
        info = plsc.get_sparse_core_info()
        return info.num_cores, info.num_subcores, info.num_lanes
    except Exception:
        return 2, 16, 16


def _build_sc_call(V, H, N, VP):
    NC, NS, L = _sc_geometry()
    NW = NC * NS
    S = N + 1
    RPW = N // NW
    RCHUNK = 8
    NM = RPW // RCHUNK
    BLKS = N // L
    WP = (S // L + 1) * L

    mesh = plsc.VectorSubcoreMesh(core_axis_name="c", subcore_axis_name="s",
                                  num_cores=NC, num_subcores=NS)

    def body(tab_hbm, idxp_hbm, out_hbm, tab_v, idx_v, ob0, ob1, sem0, sem1):
        c = lax.axis_index("c")
        s = lax.axis_index("s")
        wid = s * NC + c

        pltpu.sync_copy(tab_hbm, tab_v)

        zf = jnp.zeros((L,), jnp.float32)
        z16 = jnp.zeros((L,), jnp.int32)
        lane = lax.iota(jnp.int32, L)
        m01 = jnp.where(lane == 0, 0.0, 1.0)
        lane0 = lane == 0
        colN = jnp.full((L,), N, jnp.int32)

        tab_slices = [tab_v.at[pl.ds(h * VP, VP)] for h in range(H)]

        def fill_buf(buf, rr):
            @plsc.parallel_loop(1, BLKS, unroll=2)
            def blk(j):
                vidx = idx_v[rr, pl.ds(j * L, L)]
                for h in range(H):
                    buf[0, h, pl.ds(j * L, L)] = plsc.load_gather(
                        tab_slices[h], [vidx])

            vidx0 = idx_v[rr, pl.ds(0, L)]
            vidxN = idx_v[rr, pl.ds(N, L)]
            for h in range(H):
                v0 = plsc.load_gather(tab_slices[h], [vidx0])
                buf[0, h, pl.ds(0, L)] = v0 * m01
                vN = plsc.load_gather(tab_slices[h], [vidxN])
                plsc.store_scatter(buf, [z16, z16 + h, colN], vN,
                                   mask=lane0)

        def zero_buf(buf):
            def hloop(h, carry):
                def zblk(j, inner):
                    buf[0, h, pl.ds(j * L, L)] = zf
                    return inner

                lax.fori_loop(0, BLKS, zblk, 0)
                plsc.store_scatter(buf, [z16, z16 + h, colN], zf,
                                   mask=lane0)
                return carry

            lax.fori_loop(0, H, hloop, 0)

        base = wid * RPW

        def drain(buf, sem):
            pltpu.make_async_copy(buf, out_hbm.at[pl.ds(0, 1)], sem).wait()

        pltpu.sync_copy(idxp_hbm.at[pl.ds(pl.multiple_of(base, RCHUNK),
                                          RCHUNK)], idx_v)
        fill_buf(ob0, 0)

        @pl.when(wid == 0)
        def _z():
            zero_buf(ob0)

        pltpu.async_copy(ob0, out_hbm.at[pl.ds(base, 1)], sem0)
        fill_buf(ob1, 1)
        pltpu.async_copy(ob1, out_hbm.at[pl.ds(base + 1, 1)], sem1)

        def pair(g, carry):
            row = base + 2 * g
            rr = (2 * g) % RCHUNK

            @pl.when(rr == 0)
            def _reload():
                i0 = pl.multiple_of(row, RCHUNK)
                pltpu.sync_copy(idxp_hbm.at[pl.ds(i0, RCHUNK)], idx_v)

            drain(ob0, sem0)
            fill_buf(ob0, rr)
            pltpu.async_copy(ob0, out_hbm.at[pl.ds(row, 1)], sem0)
            drain(ob1, sem1)
            fill_buf(ob1, rr + 1)
            pltpu.async_copy(ob1, out_hbm.at[pl.ds(row + 1, 1)], sem1)
            return carry

        lax.fori_loop(1, RPW // 2, pair, 0)
        drain(ob0, sem0)
        drain(ob1, sem1)

        @pl.when(wid == NW - 1)
        def _tail():
            pltpu.sync_copy(idxp_hbm.at[pl.ds(N, 1)],
                            idx_v.at[pl.ds(0, 1)])
            fill_buf(ob0, 0)
            pltpu.sync_copy(ob0, out_hbm.at[pl.ds(N, 1)])

    call = pl.kernel(
        body,
        out_type=jax.ShapeDtypeStruct((S, H, S), jnp.float32),
        mesh=mesh,
        compiler_params=pltpu.CompilerParams(needs_layout_passes=False),
        scratch_types=[
            pltpu.VMEM((H * VP,), jnp.float32),
            pltpu.VMEM((RCHUNK, WP), jnp.int32),
            pltpu.VMEM((1, H, S), jnp.float32),
            pltpu.VMEM((1, H, S), jnp.float32),
            pltpu.SemaphoreType.DMA,
            pltpu.SemaphoreType.DMA,
        ],
    )
    return call


def kernel(relative_position_bias_table, relative_position_index, seq_len):
    V, H = relative_position_bias_table.shape
    N = relative_position_index.shape[0]
    S = N + 1
    L = 16
    WP = (S // L + 1) * L
    VP = -(-V // 128) * 128
    tab_flat = jnp.pad(relative_position_bias_table.T,
                       ((0, 0), (0, VP - V))).reshape(-1)
    idx = relative_position_index.astype(jnp.int32)
    idx_pad = jnp.pad(idx, ((1, 0), (1, WP - S)))
    call = _build_sc_call(V, H, N, VP)
    out = call(tab_flat, idx_pad)
    return jnp.transpose(out, (1, 0, 2))[None]
